# baseline (speedup 1.0000x reference)
import jax
import jax.numpy as jnp
from jax import lax
from jax.experimental import pallas as pl
from jax.experimental.pallas import tpu as pltpu
from jax.experimental.pallas import tpu_sc as plsc

N_NODES = 50000
N_ELEM = 800000
N_TILES = 32
E_PER_TEC = 25600
E_PAD = N_TILES * E_PER_TEC
CHUNK = 1024
N_CHUNKS = E_PER_TEC // CHUNK
NODE_PAD = 50176
SLICE_W = NODE_PAD // 16

i32 = jnp.int32


def _sc_body(na2, nb2, lh, eh, ah, ih, ch, sh, wh, thh, ucv, thv, out,
             tbl_w, tbl_t,
             ia0, ib0, lb0, eb0, ab0, if0, cb0, sb0,
             xa0, ya0, za0, xb0, yb0, zb0,
             ia1, ib1, lb1, eb1, ab1, if1, cb1, sb1,
             xa1, ya1, za1, xb1, yb1, zb1,
             zbuf, ucb, tcb, fx_s, fy_s, fz_s,
             sem_in0, sem_in1, sem_sc0, sem_sc1):
    core = lax.axis_index("c")
    sub = lax.axis_index("s")
    wid = sub * 2 + core
    base_tec = wid * E_PER_TEC
    himask = i32(-65536)
    one = jnp.full((16,), 1, jnp.int32)
    zero16 = jnp.zeros((16,), jnp.float32)
    zero16i = jnp.zeros((16,), jnp.int32)

    s0 = dict(ia=ia0, ib=ib0, lb=lb0, eb=eb0, ab=ab0, ifb=if0, cb=cb0, sb=sb0,
              xa=xa0, ya=ya0, za=za0, xb=xb0, yb=yb0, zb=zb0,
              sin=sem_in0, ssc=sem_sc0)
    s1 = dict(ia=ia1, ib=ib1, lb=lb1, eb=eb1, ab=ab1, ifb=if1, cb=cb1, sb=sb1,
              xa=xa1, ya=ya1, za=za1, xb=xb1, yb=yb1, zb=zb1,
              sin=sem_in1, ssc=sem_sc1)

    pltpu.sync_copy(wh, tbl_w)
    pltpu.sync_copy(thh, tbl_t)
    pltpu.sync_copy(ucv, ucb)
    pltpu.sync_copy(thv, tcb)

    def zloop(i, carry):
        zbuf[pl.ds(i * 16, 16)] = zero16
        return carry

    lax.fori_loop(i32(0), i32(64), zloop, i32(0))

    off = sub * SLICE_W
    for f_s in (fx_s, fy_s, fz_s):
        pltpu.sync_copy(zbuf, f_s.at[pl.ds(off, 1024)])
        pltpu.sync_copy(zbuf, f_s.at[pl.ds(off + 1024, 1024)])
        pltpu.sync_copy(zbuf, f_s.at[pl.ds(off + 2048, 1024)])
        pltpu.sync_copy(zbuf.at[pl.ds(0, 64)], f_s.at[pl.ds(off + 3072, 64)])
    plsc.subcore_barrier()

    ucvv = ucb[...]
    tcvv = tcb[...]

    def in_copies(s, c, fn):
        base = base_tec + c * CHUNK
        row = pl.multiple_of(lax.shift_right_logical(base, i32(7)), 8)
        sl = pl.ds(base, CHUNK)
        fn(na2.at[pl.ds(row, 8)], s["ia"], s["sin"])
        fn(nb2.at[pl.ds(row, 8)], s["ib"], s["sin"])
        fn(lh.at[sl], s["lb"], s["sin"])
        fn(eh.at[sl], s["eb"], s["sin"])
        fn(ah.at[sl], s["ab"], s["sin"])
        fn(ih.at[sl], s["ifb"], s["sin"])
        fn(ch.at[sl], s["cb"], s["sin"])
        fn(sh.at[sl], s["sb"], s["sin"])

    def issue_in(s, c):
        in_copies(s, c, lambda a, b, m: pltpu.async_copy(a, b, m))

    def wait_in(s, c):
        in_copies(s, c, lambda a, b, m: pltpu.make_async_copy(a, b, m).wait())

    def sc_copies(s, fire):
        for j in range(8):
            sl = pl.ds(j * 128, 128)
            ja = s["ia"].at[i32(j)]
            jb = s["ib"].at[i32(j)]
            for src, dst in ((s["xa"], fx_s.at[ja]), (s["ya"], fy_s.at[ja]),
                             (s["za"], fz_s.at[ja]), (s["xb"], fx_s.at[jb]),
                             (s["yb"], fy_s.at[jb]), (s["zb"], fz_s.at[jb])):
                if fire:
                    pltpu.async_copy(src.at[sl], dst, s["ssc"], add=True)
                else:
                    pltpu.make_async_copy(src.at[sl], dst, s["ssc"]).wait()

    def compute(s):
        ia, ib = s["ia"], s["ib"]
        lb, eb, ab, ifb, cb, sb = s["lb"], s["eb"], s["ab"], s["ifb"], s["cb"], s["sb"]
        xa, ya, za, xb, yb, zb = s["xa"], s["ya"], s["za"], s["xb"], s["yb"], s["zb"]

        def grp(i, c2):
            j = lax.shift_right_logical(i, i32(3))
            r = lax.bitwise_and(i, i32(7)) * 16
            idx_a = ia[j, pl.ds(r, 16)]
            idx_b = ib[j, pl.ds(r, 16)]
            w_a = plsc.load_gather(tbl_w, [idx_a])
            w_b = plsc.load_gather(tbl_w, [idx_b])
            tp_a = plsc.load_gather(tbl_t, [lax.shift_right_logical(idx_a, one)])
            tp_b = plsc.load_gather(tbl_t, [lax.shift_right_logical(idx_b, one)])
            ux_a = plsc.bitcast(w_a << 16, jnp.float32) * ucvv
            uz_a = plsc.bitcast(w_a & himask, jnp.float32) * ucvv
            ux_b = plsc.bitcast(w_b << 16, jnp.float32) * ucvv
            uz_b = plsc.bitcast(w_b & himask, jnp.float32) * ucvv
            odd_a = lax.bitwise_and(idx_a, one) == one
            odd_b = lax.bitwise_and(idx_b, one) == one
            th_a = plsc.bitcast(
                jnp.where(odd_a, tp_a & himask, tp_a << 16), jnp.float32) * tcvv
            th_b = plsc.bitcast(
                jnp.where(odd_b, tp_b & himask, tp_b << 16), jnp.float32) * tcvv
            o = i * 16
            lv = lb[pl.ds(o, 16)]
            ev = eb[pl.ds(o, 16)]
            av = ab[pl.ds(o, 16)]
            iv = ifb[pl.ds(o, 16)]
            cv = cb[pl.ds(o, 16)]
            sv = sb[pl.ds(o, 16)]
            rcp = 1.0 / lv
            eal = ev * av * rcp
            eil = ev * iv * rcp
            ei2 = eil * rcp
            ei3 = ei2 * rcp
            u_al = cv * ux_a + sv * uz_a
            w_al = cv * uz_a - sv * ux_a
            u_bl = cv * ux_b + sv * uz_b
            w_bl = cv * uz_b - sv * ux_b
            f0 = eal * (u_al - u_bl)
            dw = w_al - w_bl
            f1 = 12.0 * ei3 * dw - 6.0 * ei2 * (th_a + th_b)
            f2 = 6.0 * ei2 * dw - eil * (4.0 * th_a + 2.0 * th_b)
            f5 = 6.0 * ei2 * dw - eil * (2.0 * th_a + 4.0 * th_b)
            gx = cv * f0 - sv * f1
            gy = sv * f0 + cv * f1
            xa[pl.ds(o, 16)] = gx
            ya[pl.ds(o, 16)] = gy
            za[pl.ds(o, 16)] = -f2
            xb[pl.ds(o, 16)] = -gx
            yb[pl.ds(o, 16)] = -gy
            zb[pl.ds(o, 16)] = -f5
            return c2

        lax.fori_loop(i32(0), i32(64), grp, i32(0))

    for j in range(8):
        for r in range(0, 128, 16):
            ia1[i32(j), pl.ds(r, 16)] = zero16i
            ib1[i32(j), pl.ds(r, 16)] = zero16i

    def vzloop(i, carry):
        o = i * 16
        for buf in (xa1, ya1, za1, xb1, yb1, zb1):
            buf[pl.ds(o, 16)] = zero16
        return carry

    lax.fori_loop(i32(0), i32(64), vzloop, i32(0))
    sc_copies(s1, fire=True)

    issue_in(s0, i32(0))

    def pair(t, carry):
        c0 = t * 2
        wait_in(s0, c0)
        compute(s0)
        sc_copies(s0, fire=True)
        sc_copies(s1, fire=False)
        issue_in(s1, c0 + 1)
        c1 = c0 + 1
        wait_in(s1, c1)
        compute(s1)
        sc_copies(s1, fire=True)
        sc_copies(s0, fire=False)
        issue_in(s0, c1 + 1)
        return carry

    lax.fori_loop(i32(0), i32((N_CHUNKS - 1) // 2), pair, i32(0))

    ct = i32(N_CHUNKS - 1)
    wait_in(s0, ct)
    compute(s0)
    sc_copies(s0, fire=True)
    sc_copies(s1, fire=False)
    sc_copies(s0, fire=False)
    plsc.subcore_barrier()

    obase = core * (3 * NODE_PAD) + off
    for comp, f_s in enumerate((fx_s, fy_s, fz_s)):
        dst0 = obase + comp * NODE_PAD
        for p, w in ((0, 1024), (1024, 1024), (2048, 1024), (3072, 64)):
            pltpu.sync_copy(f_s.at[pl.ds(off + p, w)], zbuf.at[pl.ds(0, w)])
            pltpu.sync_copy(zbuf.at[pl.ds(0, w)],
                            out.at[pl.ds(pl.multiple_of(dst0 + p, 8), w)])


def _tc_body(pred_ref, scale_ref, p_ref, fet_ref, bct_ref, up_ref, loss_ref):
    up_ref[...] = pred_ref[...] * scale_ref[...]
    s = p_ref[0] + p_ref[1]
    mask = 1.0 - bct_ref[...]
    fe = fet_ref[...]
    rm = (s - fe) * mask
    fm = fe * mask
    num = jnp.sum(rm * rm)
    den = jnp.maximum(jnp.sum(fm * fm), 1e-30)
    loss_ref[...] = jnp.reshape(num / den, (1, 1))


def kernel(pred_raw, u_c, theta_c, connectivity, elem_lengths, prop_E, prop_A,
           prop_I22, elem_directions, F_ext, bc_disp, bc_rot):
    pad = E_PAD - N_ELEM
    conn = connectivity.astype(jnp.int32)
    na = jnp.pad(conn[:, 0], (0, pad))
    nb = jnp.pad(conn[:, 1], (0, pad))
    na2 = na.reshape(E_PAD // 128, 128)
    nb2 = nb.reshape(E_PAD // 128, 128)
    l_p = jnp.pad(elem_lengths.astype(jnp.float32), (0, pad), constant_values=1.0)
    e_p = jnp.pad(prop_E.astype(jnp.float32), (0, pad))
    a_p = jnp.pad(prop_A.astype(jnp.float32), (0, pad))
    i_p = jnp.pad(prop_I22.astype(jnp.float32), (0, pad))
    c_p = jnp.pad(elem_directions[:, 0].astype(jnp.float32), (0, pad))
    s_p = jnp.pad(elem_directions[:, 2].astype(jnp.float32), (0, pad))

    bx = lax.bitcast_convert_type(
        pred_raw[:, 0].astype(jnp.bfloat16), jnp.uint16).astype(jnp.uint32)
    bz = lax.bitcast_convert_type(
        pred_raw[:, 1].astype(jnp.bfloat16), jnp.uint16).astype(jnp.uint32)
    word = lax.bitcast_convert_type(bx | (bz << 16), jnp.int32)
    bt = lax.bitcast_convert_type(
        pred_raw[:, 2].astype(jnp.bfloat16), jnp.uint16).astype(jnp.uint32)
    bt2 = jnp.pad(bt, (0, NODE_PAD - N_NODES)).reshape(NODE_PAD // 2, 2)
    thcol = lax.bitcast_convert_type(bt2[:, 0] | (bt2[:, 1] << 16), jnp.int32)
    ucv = jnp.full((16,), u_c[0], jnp.float32)
    thv = jnp.full((16,), theta_c[0], jnp.float32)

    mesh = plsc.VectorSubcoreMesh(core_axis_name="c", subcore_axis_name="s")
    f32 = jnp.float32
    vm = pltpu.VMEM
    sc_call = pl.kernel(
        _sc_body,
        out_type=jax.ShapeDtypeStruct((2 * 3 * NODE_PAD,), f32),
        mesh=mesh,
        compiler_params=pltpu.CompilerParams(needs_layout_passes=False),
        scratch_types=[
            vm((N_NODES,), jnp.int32),
            vm((NODE_PAD // 2,), jnp.int32),
            vm((8, 128), jnp.int32), vm((8, 128), jnp.int32),
            vm((CHUNK,), f32), vm((CHUNK,), f32), vm((CHUNK,), f32),
            vm((CHUNK,), f32), vm((CHUNK,), f32), vm((CHUNK,), f32),
            vm((CHUNK,), f32), vm((CHUNK,), f32), vm((CHUNK,), f32),
            vm((CHUNK,), f32), vm((CHUNK,), f32), vm((CHUNK,), f32),
            vm((8, 128), jnp.int32), vm((8, 128), jnp.int32),
            vm((CHUNK,), f32), vm((CHUNK,), f32), vm((CHUNK,), f32),
            vm((CHUNK,), f32), vm((CHUNK,), f32), vm((CHUNK,), f32),
            vm((CHUNK,), f32), vm((CHUNK,), f32), vm((CHUNK,), f32),
            vm((CHUNK,), f32), vm((CHUNK,), f32), vm((CHUNK,), f32),
            vm((1024,), f32),
            vm((16,), f32),
            vm((16,), f32),
            pltpu.VMEM_SHARED((NODE_PAD,), f32),
            pltpu.VMEM_SHARED((NODE_PAD,), f32),
            pltpu.VMEM_SHARED((NODE_PAD,), f32),
            pltpu.SemaphoreType.DMA, pltpu.SemaphoreType.DMA,
            pltpu.SemaphoreType.DMA, pltpu.SemaphoreType.DMA,
        ],
    )
    partials = sc_call(na2, nb2, l_p, e_p, a_p, i_p, c_p, s_p, word, thcol,
                       ucv, thv).reshape(2, 3, NODE_PAD)

    scale3 = jnp.concatenate([u_c, u_c, theta_c]).astype(f32).reshape(3, 1)
    pred_t = jnp.pad(pred_raw.astype(f32).T, ((0, 0), (0, NODE_PAD - N_NODES)))
    fet = jnp.pad(F_ext.astype(f32).T, ((0, 0), (0, NODE_PAD - N_NODES)))
    bct = jnp.pad(
        jnp.concatenate([bc_disp, bc_disp, bc_rot], axis=1).astype(f32).T,
        ((0, 0), (0, NODE_PAD - N_NODES)))

    u_phys_t, loss2d = pl.pallas_call(
        _tc_body,
        out_shape=[
            jax.ShapeDtypeStruct((3, NODE_PAD), f32),
            jax.ShapeDtypeStruct((1, 1), f32),
        ],
    )(pred_t, scale3, partials, fet, bct)
    return loss2d[0, 0], pred_raw, u_phys_t[:, :N_NODES].T


# speedup vs baseline: 1.0365x; 1.0365x over previous
import jax
import jax.numpy as jnp
from jax import lax
from jax.experimental import pallas as pl
from jax.experimental.pallas import tpu as pltpu
from jax.experimental.pallas import tpu_sc as plsc

N_NODES = 50000
N_ELEM = 800000
N_TILES = 32
E_PER_TEC = 25600
E_PAD = N_TILES * E_PER_TEC
CHUNK = 1024
N_CHUNKS = E_PER_TEC // CHUNK
NODE_PAD = 50176
SLICE_W = NODE_PAD // 16

i32 = jnp.int32


def _sc_body(nah, nbh, lh, eh, ah, ih, ch, sh, wh, thh, ucv, thv, out,
             tbl_w, tbl_t,
             ia0, ib0, lb0, eb0, ab0, if0, cb0, sb0,
             xa0, ya0, za0, xb0, yb0, zb0,
             ia1, ib1, lb1, eb1, ab1, if1, cb1, sb1,
             xa1, ya1, za1, xb1, yb1, zb1,
             zbuf, ucb, tcb, fx_s, fy_s, fz_s,
             sem_in0, sem_in1, sem_sc0, sem_sc1):
    core = lax.axis_index("c")
    sub = lax.axis_index("s")
    wid = sub * 2 + core
    base_tec = wid * E_PER_TEC
    himask = i32(-65536)
    one = jnp.full((16,), 1, jnp.int32)
    zero16 = jnp.zeros((16,), jnp.float32)
    zero16i = jnp.zeros((16,), jnp.int32)

    s0 = dict(ia=ia0, ib=ib0, lb=lb0, eb=eb0, ab=ab0, ifb=if0, cb=cb0, sb=sb0,
              xa=xa0, ya=ya0, za=za0, xb=xb0, yb=yb0, zb=zb0,
              sin=sem_in0, ssc=sem_sc0)
    s1 = dict(ia=ia1, ib=ib1, lb=lb1, eb=eb1, ab=ab1, ifb=if1, cb=cb1, sb=sb1,
              xa=xa1, ya=ya1, za=za1, xb=xb1, yb=yb1, zb=zb1,
              sin=sem_in1, ssc=sem_sc1)

    pltpu.sync_copy(wh, tbl_w)
    pltpu.sync_copy(thh, tbl_t)
    pltpu.sync_copy(ucv, ucb)
    pltpu.sync_copy(thv, tcb)

    def zloop(i, carry):
        zbuf[pl.ds(i * 16, 16)] = zero16
        return carry

    lax.fori_loop(i32(0), i32(64), zloop, i32(0))

    off = sub * SLICE_W
    for f_s in (fx_s, fy_s, fz_s):
        pltpu.sync_copy(zbuf, f_s.at[pl.ds(off, 1024)])
        pltpu.sync_copy(zbuf, f_s.at[pl.ds(off + 1024, 1024)])
        pltpu.sync_copy(zbuf, f_s.at[pl.ds(off + 2048, 1024)])
        pltpu.sync_copy(zbuf.at[pl.ds(0, 64)], f_s.at[pl.ds(off + 3072, 64)])
    plsc.subcore_barrier()

    ucvv = ucb[...]
    tcvv = tcb[...]

    def in_copies(s, c, fn):
        base = pl.multiple_of(base_tec + c * CHUNK, 8)
        sl = pl.ds(base, CHUNK)
        fn(nah.at[sl], s["ia"], s["sin"])
        fn(nbh.at[sl], s["ib"], s["sin"])
        fn(lh.at[sl], s["lb"], s["sin"])
        fn(eh.at[sl], s["eb"], s["sin"])
        fn(ah.at[sl], s["ab"], s["sin"])
        fn(ih.at[sl], s["ifb"], s["sin"])
        fn(ch.at[sl], s["cb"], s["sin"])
        fn(sh.at[sl], s["sb"], s["sin"])

    def issue_in(s, c):
        in_copies(s, c, lambda a, b, m: pltpu.async_copy(a, b, m))

    def wait_in(s, c):
        in_copies(s, c, lambda a, b, m: pltpu.make_async_copy(a, b, m).wait())

    def sc_copies(s, fire):
        ja = s["ia"]
        jb = s["ib"]
        for vsrc, dst in ((s["xa"], fx_s.at[ja]), (s["ya"], fy_s.at[ja]),
                          (s["za"], fz_s.at[ja]), (s["xb"], fx_s.at[jb]),
                          (s["yb"], fy_s.at[jb]), (s["zb"], fz_s.at[jb])):
            if fire:
                pltpu.async_copy(vsrc, dst, s["ssc"], add=True)
            else:
                pltpu.make_async_copy(vsrc, dst, s["ssc"]).wait()

    def compute(s):
        ia, ib = s["ia"], s["ib"]
        lb, eb, ab, ifb, cb, sb = s["lb"], s["eb"], s["ab"], s["ifb"], s["cb"], s["sb"]
        xa, ya, za, xb, yb, zb = s["xa"], s["ya"], s["za"], s["xb"], s["yb"], s["zb"]

        def grp(i, c2):
            o = i * 16
            idx_a = ia[pl.ds(o, 16)]
            idx_b = ib[pl.ds(o, 16)]
            w_a = plsc.load_gather(tbl_w, [idx_a])
            w_b = plsc.load_gather(tbl_w, [idx_b])
            tp_a = plsc.load_gather(tbl_t, [lax.shift_right_logical(idx_a, one)])
            tp_b = plsc.load_gather(tbl_t, [lax.shift_right_logical(idx_b, one)])
            ux_a = plsc.bitcast(w_a << 16, jnp.float32) * ucvv
            uz_a = plsc.bitcast(w_a & himask, jnp.float32) * ucvv
            ux_b = plsc.bitcast(w_b << 16, jnp.float32) * ucvv
            uz_b = plsc.bitcast(w_b & himask, jnp.float32) * ucvv
            odd_a = lax.bitwise_and(idx_a, one) == one
            odd_b = lax.bitwise_and(idx_b, one) == one
            th_a = plsc.bitcast(
                jnp.where(odd_a, tp_a & himask, tp_a << 16), jnp.float32) * tcvv
            th_b = plsc.bitcast(
                jnp.where(odd_b, tp_b & himask, tp_b << 16), jnp.float32) * tcvv
            lv = lb[pl.ds(o, 16)]
            ev = eb[pl.ds(o, 16)]
            av = ab[pl.ds(o, 16)]
            iv = ifb[pl.ds(o, 16)]
            cv = cb[pl.ds(o, 16)]
            sv = sb[pl.ds(o, 16)]
            rcp = 1.0 / lv
            eal = ev * av * rcp
            eil = ev * iv * rcp
            ei2 = eil * rcp
            ei3 = ei2 * rcp
            u_al = cv * ux_a + sv * uz_a
            w_al = cv * uz_a - sv * ux_a
            u_bl = cv * ux_b + sv * uz_b
            w_bl = cv * uz_b - sv * ux_b
            f0 = eal * (u_al - u_bl)
            dw = w_al - w_bl
            f1 = 12.0 * ei3 * dw - 6.0 * ei2 * (th_a + th_b)
            f2 = 6.0 * ei2 * dw - eil * (4.0 * th_a + 2.0 * th_b)
            f5 = 6.0 * ei2 * dw - eil * (2.0 * th_a + 4.0 * th_b)
            gx = cv * f0 - sv * f1
            gy = sv * f0 + cv * f1
            xa[pl.ds(o, 16)] = gx
            ya[pl.ds(o, 16)] = gy
            za[pl.ds(o, 16)] = -f2
            xb[pl.ds(o, 16)] = -gx
            yb[pl.ds(o, 16)] = -gy
            zb[pl.ds(o, 16)] = -f5
            return c2

        lax.fori_loop(i32(0), i32(64), grp, i32(0))

    def vzloop(i, carry):
        o = i * 16
        ia1[pl.ds(o, 16)] = zero16i
        ib1[pl.ds(o, 16)] = zero16i
        for buf in (xa1, ya1, za1, xb1, yb1, zb1):
            buf[pl.ds(o, 16)] = zero16
        return carry

    lax.fori_loop(i32(0), i32(64), vzloop, i32(0))
    sc_copies(s1, fire=True)

    issue_in(s0, i32(0))

    def pair(t, carry):
        c0 = t * 2
        wait_in(s0, c0)
        compute(s0)
        sc_copies(s0, fire=True)
        sc_copies(s1, fire=False)
        issue_in(s1, c0 + 1)
        c1 = c0 + 1
        wait_in(s1, c1)
        compute(s1)
        sc_copies(s1, fire=True)
        sc_copies(s0, fire=False)
        issue_in(s0, c1 + 1)
        return carry

    lax.fori_loop(i32(0), i32((N_CHUNKS - 1) // 2), pair, i32(0))

    ct = i32(N_CHUNKS - 1)
    wait_in(s0, ct)
    compute(s0)
    sc_copies(s0, fire=True)
    sc_copies(s1, fire=False)
    sc_copies(s0, fire=False)
    plsc.subcore_barrier()

    obase = core * (3 * NODE_PAD) + off
    for comp, f_s in enumerate((fx_s, fy_s, fz_s)):
        dst0 = obase + comp * NODE_PAD
        for p, w in ((0, 1024), (1024, 1024), (2048, 1024), (3072, 64)):
            pltpu.sync_copy(f_s.at[pl.ds(off + p, w)], zbuf.at[pl.ds(0, w)])
            pltpu.sync_copy(zbuf.at[pl.ds(0, w)],
                            out.at[pl.ds(pl.multiple_of(dst0 + p, 8), w)])


def _tc_body(pred_ref, scale_ref, p_ref, fet_ref, bct_ref, up_ref, loss_ref):
    up_ref[...] = pred_ref[...] * scale_ref[...]
    s = p_ref[0] + p_ref[1]
    mask = 1.0 - bct_ref[...]
    fe = fet_ref[...]
    rm = (s - fe) * mask
    fm = fe * mask
    num = jnp.sum(rm * rm)
    den = jnp.maximum(jnp.sum(fm * fm), 1e-30)
    loss_ref[...] = jnp.reshape(num / den, (1, 1))


def kernel(pred_raw, u_c, theta_c, connectivity, elem_lengths, prop_E, prop_A,
           prop_I22, elem_directions, F_ext, bc_disp, bc_rot):
    pad = E_PAD - N_ELEM
    conn = connectivity.astype(jnp.int32)
    na = jnp.pad(conn[:, 0], (0, pad))
    nb = jnp.pad(conn[:, 1], (0, pad))

    l_p = jnp.pad(elem_lengths.astype(jnp.float32), (0, pad), constant_values=1.0)
    e_p = jnp.pad(prop_E.astype(jnp.float32), (0, pad))
    a_p = jnp.pad(prop_A.astype(jnp.float32), (0, pad))
    i_p = jnp.pad(prop_I22.astype(jnp.float32), (0, pad))
    c_p = jnp.pad(elem_directions[:, 0].astype(jnp.float32), (0, pad))
    s_p = jnp.pad(elem_directions[:, 2].astype(jnp.float32), (0, pad))

    bx = lax.bitcast_convert_type(
        pred_raw[:, 0].astype(jnp.bfloat16), jnp.uint16).astype(jnp.uint32)
    bz = lax.bitcast_convert_type(
        pred_raw[:, 1].astype(jnp.bfloat16), jnp.uint16).astype(jnp.uint32)
    word = lax.bitcast_convert_type(bx | (bz << 16), jnp.int32)
    bt = lax.bitcast_convert_type(
        pred_raw[:, 2].astype(jnp.bfloat16), jnp.uint16).astype(jnp.uint32)
    bt2 = jnp.pad(bt, (0, NODE_PAD - N_NODES)).reshape(NODE_PAD // 2, 2)
    thcol = lax.bitcast_convert_type(bt2[:, 0] | (bt2[:, 1] << 16), jnp.int32)
    ucv = jnp.full((16,), u_c[0], jnp.float32)
    thv = jnp.full((16,), theta_c[0], jnp.float32)

    mesh = plsc.VectorSubcoreMesh(core_axis_name="c", subcore_axis_name="s")
    f32 = jnp.float32
    vm = pltpu.VMEM
    sc_call = pl.kernel(
        _sc_body,
        out_type=jax.ShapeDtypeStruct((2 * 3 * NODE_PAD,), f32),
        mesh=mesh,
        compiler_params=pltpu.CompilerParams(needs_layout_passes=False),
        scratch_types=[
            vm((N_NODES,), jnp.int32),
            vm((NODE_PAD // 2,), jnp.int32),
            vm((CHUNK,), jnp.int32), vm((CHUNK,), jnp.int32),
            vm((CHUNK,), f32), vm((CHUNK,), f32), vm((CHUNK,), f32),
            vm((CHUNK,), f32), vm((CHUNK,), f32), vm((CHUNK,), f32),
            vm((CHUNK,), f32), vm((CHUNK,), f32), vm((CHUNK,), f32),
            vm((CHUNK,), f32), vm((CHUNK,), f32), vm((CHUNK,), f32),
            vm((CHUNK,), jnp.int32), vm((CHUNK,), jnp.int32),
            vm((CHUNK,), f32), vm((CHUNK,), f32), vm((CHUNK,), f32),
            vm((CHUNK,), f32), vm((CHUNK,), f32), vm((CHUNK,), f32),
            vm((CHUNK,), f32), vm((CHUNK,), f32), vm((CHUNK,), f32),
            vm((CHUNK,), f32), vm((CHUNK,), f32), vm((CHUNK,), f32),
            vm((1024,), f32),
            vm((16,), f32),
            vm((16,), f32),
            pltpu.VMEM_SHARED((NODE_PAD,), f32),
            pltpu.VMEM_SHARED((NODE_PAD,), f32),
            pltpu.VMEM_SHARED((NODE_PAD,), f32),
            pltpu.SemaphoreType.DMA, pltpu.SemaphoreType.DMA,
            pltpu.SemaphoreType.DMA, pltpu.SemaphoreType.DMA,
        ],
    )
    partials = sc_call(na, nb, l_p, e_p, a_p, i_p, c_p, s_p, word, thcol,
                       ucv, thv).reshape(2, 3, NODE_PAD)

    scale3 = jnp.concatenate([u_c, u_c, theta_c]).astype(f32).reshape(3, 1)
    pred_t = jnp.pad(pred_raw.astype(f32).T, ((0, 0), (0, NODE_PAD - N_NODES)))
    fet = jnp.pad(F_ext.astype(f32).T, ((0, 0), (0, NODE_PAD - N_NODES)))
    bct = jnp.pad(
        jnp.concatenate([bc_disp, bc_disp, bc_rot], axis=1).astype(f32).T,
        ((0, 0), (0, NODE_PAD - N_NODES)))

    u_phys_t, loss2d = pl.pallas_call(
        _tc_body,
        out_shape=[
            jax.ShapeDtypeStruct((3, NODE_PAD), f32),
            jax.ShapeDtypeStruct((1, 1), f32),
        ],
    )(pred_t, scale3, partials, fet, bct)
    return loss2d[0, 0], pred_raw, u_phys_t[:, :N_NODES].T


# speedup vs baseline: 1.2232x; 1.1801x over previous
import jax
import jax.numpy as jnp
from jax import lax
from jax.experimental import pallas as pl
from jax.experimental.pallas import tpu as pltpu
from jax.experimental.pallas import tpu_sc as plsc

N_NODES = 50000
N_ELEM = 800000
N_TILES = 32
E_PER_TEC = 25600
E_PAD = N_TILES * E_PER_TEC
CHUNK = 1024
N_CHUNKS = E_PER_TEC // CHUNK
NODE_PAD = 50176
SLICE_W = NODE_PAD // 16

i32 = jnp.int32


def _sc_body(na2, nb2, lh, eh, ah, ih, ch, sh, wh, thh, ucv, thv, out,
             tbl_w, tbl_t, ia2, ib2, lb, eb, ab, ibf, cb, sb,
             vxa, vya, vza, vxb, vyb, vzb, zb, ucb, tcb, fx_s, fy_s, fz_s,
             sem_in, sem_sc):
    core = lax.axis_index("c")
    sub = lax.axis_index("s")
    wid = sub * 2 + core

    pltpu.sync_copy(wh, tbl_w)
    pltpu.sync_copy(thh, tbl_t)
    pltpu.sync_copy(ucv, ucb)
    pltpu.sync_copy(thv, tcb)

    zero16 = jnp.zeros((16,), jnp.float32)

    def zloop(i, carry):
        zb[pl.ds(i * 16, 16)] = zero16
        return carry

    lax.fori_loop(i32(0), i32(64), zloop, i32(0))

    off = sub * SLICE_W
    for f_s in (fx_s, fy_s, fz_s):
        pltpu.sync_copy(zb, f_s.at[pl.ds(off, 1024)])
        pltpu.sync_copy(zb, f_s.at[pl.ds(off + 1024, 1024)])
        pltpu.sync_copy(zb, f_s.at[pl.ds(off + 2048, 1024)])
        pltpu.sync_copy(zb.at[pl.ds(0, 64)], f_s.at[pl.ds(off + 3072, 64)])
    plsc.subcore_barrier()

    ucvv = ucb[...]
    tcvv = tcb[...]
    base_tec = wid * E_PER_TEC
    himask = i32(-65536)

    def chunk_body(k, carry):
        base = base_tec + k * CHUNK
        row = pl.multiple_of(lax.shift_right_logical(base, i32(7)), 8)
        sl_in = pl.ds(base, CHUNK)
        cps = [
            pltpu.async_copy(na2.at[pl.ds(row, 8)], ia2, sem_in),
            pltpu.async_copy(nb2.at[pl.ds(row, 8)], ib2, sem_in),
            pltpu.async_copy(lh.at[sl_in], lb, sem_in),
            pltpu.async_copy(eh.at[sl_in], eb, sem_in),
            pltpu.async_copy(ah.at[sl_in], ab, sem_in),
            pltpu.async_copy(ih.at[sl_in], ibf, sem_in),
            pltpu.async_copy(ch.at[sl_in], cb, sem_in),
            pltpu.async_copy(sh.at[sl_in], sb, sem_in),
        ]
        for cp in cps:
            cp.wait()

        def grp(i, c2):
            j = lax.shift_right_logical(i, i32(3))
            r = lax.bitwise_and(i, i32(7)) * 16
            idx_a = ia2[j, pl.ds(r, 16)]
            idx_b = ib2[j, pl.ds(r, 16)]
            w_a = plsc.load_gather(tbl_w, [idx_a])
            w_b = plsc.load_gather(tbl_w, [idx_b])
            t_a = plsc.load_gather(tbl_t, [idx_a])
            t_b = plsc.load_gather(tbl_t, [idx_b])
            ux_a = plsc.bitcast(w_a << 16, jnp.float32) * ucvv
            uz_a = plsc.bitcast(w_a & himask, jnp.float32) * ucvv
            ux_b = plsc.bitcast(w_b << 16, jnp.float32) * ucvv
            uz_b = plsc.bitcast(w_b & himask, jnp.float32) * ucvv
            th_a = t_a * tcvv
            th_b = t_b * tcvv
            o = i * 16
            lv = lb[pl.ds(o, 16)]
            ev = eb[pl.ds(o, 16)]
            av = ab[pl.ds(o, 16)]
            iv = ibf[pl.ds(o, 16)]
            cv = cb[pl.ds(o, 16)]
            sv = sb[pl.ds(o, 16)]
            rcp = 1.0 / lv
            eal = ev * av * rcp
            eil = ev * iv * rcp
            ei2 = eil * rcp
            ei3 = ei2 * rcp
            u_al = cv * ux_a + sv * uz_a
            w_al = cv * uz_a - sv * ux_a
            u_bl = cv * ux_b + sv * uz_b
            w_bl = cv * uz_b - sv * ux_b
            f0 = eal * (u_al - u_bl)
            dw = w_al - w_bl
            f1 = 12.0 * ei3 * dw - 6.0 * ei2 * (th_a + th_b)
            f2 = 6.0 * ei2 * dw - eil * (4.0 * th_a + 2.0 * th_b)
            f5 = 6.0 * ei2 * dw - eil * (2.0 * th_a + 4.0 * th_b)
            gx = cv * f0 - sv * f1
            gy = sv * f0 + cv * f1
            vxa[pl.ds(o, 16)] = gx
            vya[pl.ds(o, 16)] = gy
            vza[pl.ds(o, 16)] = -f2
            vxb[pl.ds(o, 16)] = -gx
            vyb[pl.ds(o, 16)] = -gy
            vzb[pl.ds(o, 16)] = -f5
            return c2

        lax.fori_loop(i32(0), i32(64), grp, i32(0))

        scs = []
        for j in range(8):
            sl = pl.ds(j * 128, 128)
            ja = ia2.at[i32(j)]
            jb = ib2.at[i32(j)]
            scs.append(pltpu.async_copy(vxa.at[sl], fx_s.at[ja], sem_sc, add=True))
            scs.append(pltpu.async_copy(vya.at[sl], fy_s.at[ja], sem_sc, add=True))
            scs.append(pltpu.async_copy(vza.at[sl], fz_s.at[ja], sem_sc, add=True))
            scs.append(pltpu.async_copy(vxb.at[sl], fx_s.at[jb], sem_sc, add=True))
            scs.append(pltpu.async_copy(vyb.at[sl], fy_s.at[jb], sem_sc, add=True))
            scs.append(pltpu.async_copy(vzb.at[sl], fz_s.at[jb], sem_sc, add=True))
        for cp in scs:
            cp.wait()
        return carry

    lax.fori_loop(i32(0), i32(N_CHUNKS), chunk_body, i32(0))
    plsc.subcore_barrier()

    obase = core * (3 * NODE_PAD) + off
    for comp, f_s in enumerate((fx_s, fy_s, fz_s)):
        dst0 = obase + comp * NODE_PAD
        for p, w in ((0, 1024), (1024, 1024), (2048, 1024), (3072, 64)):
            pltpu.sync_copy(f_s.at[pl.ds(off + p, w)], zb.at[pl.ds(0, w)])
            pltpu.sync_copy(zb.at[pl.ds(0, w)],
                            out.at[pl.ds(pl.multiple_of(dst0 + p, 8), w)])


def _tc_body(pred_ref, scale_ref, p_ref, fet_ref, bct_ref, up_ref, loss_ref):
    up_ref[...] = pred_ref[...] * scale_ref[...]
    s = p_ref[0] + p_ref[1]
    mask = 1.0 - bct_ref[...]
    fe = fet_ref[...]
    rm = (s - fe) * mask
    fm = fe * mask
    num = jnp.sum(rm * rm)
    den = jnp.maximum(jnp.sum(fm * fm), 1e-30)
    loss_ref[...] = jnp.reshape(num / den, (1, 1))


def kernel(pred_raw, u_c, theta_c, connectivity, elem_lengths, prop_E, prop_A,
           prop_I22, elem_directions, F_ext, bc_disp, bc_rot):
    pad = E_PAD - N_ELEM
    conn = connectivity.astype(jnp.int32)
    na = jnp.pad(conn[:, 0], (0, pad))
    nb = jnp.pad(conn[:, 1], (0, pad))
    na2 = na.reshape(E_PAD // 128, 128)
    nb2 = nb.reshape(E_PAD // 128, 128)
    l_p = jnp.pad(elem_lengths.astype(jnp.float32), (0, pad), constant_values=1.0)
    e_p = jnp.pad(prop_E.astype(jnp.float32), (0, pad))
    a_p = jnp.pad(prop_A.astype(jnp.float32), (0, pad))
    i_p = jnp.pad(prop_I22.astype(jnp.float32), (0, pad))
    c_p = jnp.pad(elem_directions[:, 0].astype(jnp.float32), (0, pad))
    s_p = jnp.pad(elem_directions[:, 2].astype(jnp.float32), (0, pad))

    bx = lax.bitcast_convert_type(
        pred_raw[:, 0].astype(jnp.bfloat16), jnp.uint16).astype(jnp.uint32)
    bz = lax.bitcast_convert_type(
        pred_raw[:, 1].astype(jnp.bfloat16), jnp.uint16).astype(jnp.uint32)
    word = lax.bitcast_convert_type(bx | (bz << 16), jnp.int32)
    thcol = pred_raw[:, 2].astype(jnp.float32)
    ucv = jnp.full((16,), u_c[0], jnp.float32)
    thv = jnp.full((16,), theta_c[0], jnp.float32)

    mesh = plsc.VectorSubcoreMesh(core_axis_name="c", subcore_axis_name="s")
    f32 = jnp.float32
    sc_call = pl.kernel(
        _sc_body,
        out_type=jax.ShapeDtypeStruct((2 * 3 * NODE_PAD,), f32),
        mesh=mesh,
        compiler_params=pltpu.CompilerParams(needs_layout_passes=False),
        scratch_types=[
            pltpu.VMEM((N_NODES,), jnp.int32),
            pltpu.VMEM((N_NODES,), f32),
            pltpu.VMEM((8, 128), jnp.int32),
            pltpu.VMEM((8, 128), jnp.int32),
            pltpu.VMEM((CHUNK,), f32),
            pltpu.VMEM((CHUNK,), f32),
            pltpu.VMEM((CHUNK,), f32),
            pltpu.VMEM((CHUNK,), f32),
            pltpu.VMEM((CHUNK,), f32),
            pltpu.VMEM((CHUNK,), f32),
            pltpu.VMEM((CHUNK,), f32),
            pltpu.VMEM((CHUNK,), f32),
            pltpu.VMEM((CHUNK,), f32),
            pltpu.VMEM((CHUNK,), f32),
            pltpu.VMEM((CHUNK,), f32),
            pltpu.VMEM((CHUNK,), f32),
            pltpu.VMEM((1024,), f32),
            pltpu.VMEM((16,), f32),
            pltpu.VMEM((16,), f32),
            pltpu.VMEM_SHARED((NODE_PAD,), f32),
            pltpu.VMEM_SHARED((NODE_PAD,), f32),
            pltpu.VMEM_SHARED((NODE_PAD,), f32),
            pltpu.SemaphoreType.DMA,
            pltpu.SemaphoreType.DMA,
        ],
    )
    partials = sc_call(na2, nb2, l_p, e_p, a_p, i_p, c_p, s_p, word, thcol,
                       ucv, thv).reshape(2, 3, NODE_PAD)

    scale3 = jnp.concatenate([u_c, u_c, theta_c]).astype(f32).reshape(3, 1)
    pred_t = jnp.pad(pred_raw.astype(f32).T, ((0, 0), (0, NODE_PAD - N_NODES)))
    fet = jnp.pad(F_ext.astype(f32).T, ((0, 0), (0, NODE_PAD - N_NODES)))
    bct = jnp.pad(
        jnp.concatenate([bc_disp, bc_disp, bc_rot], axis=1).astype(f32).T,
        ((0, 0), (0, NODE_PAD - N_NODES)))

    u_phys_t, loss2d = pl.pallas_call(
        _tc_body,
        out_shape=[
            jax.ShapeDtypeStruct((3, NODE_PAD), f32),
            jax.ShapeDtypeStruct((1, 1), f32),
        ],
    )(pred_t, scale3, partials, fet, bct)
    return loss2d[0, 0], pred_raw, u_phys_t[:, :N_NODES].T
